# baseline (speedup 1.0000x reference)
import functools

import jax
import jax.numpy as jnp
from jax import lax
from jax.experimental import pallas as pl
from jax.experimental.pallas import tpu as pltpu
from jax.experimental.pallas import tpu_sc as plsc

BATCH, SEQ, DIM = 4, 4096, 2048
ROWS = BATCH * SEQ
NC, NS = 2, 16
NW = NC * NS
ROWS_PER_W = ROWS // NW
CHUNK = 16
LANES = 16


def _shuffle_body(x_hbm, perm_hbm, out_hbm, perm_v, in_v, out_v):
    wid = lax.axis_index("s") * NC + lax.axis_index("c")
    base = wid * (ROWS_PER_W * DIM)
    pltpu.sync_copy(perm_hbm, perm_v)

    def chunk_body(c, carry):
        off = base + c * (CHUNK * DIM)
        pltpu.sync_copy(x_hbm.at[pl.ds(off, CHUNK * DIM)], in_v)
        for j in range(DIM // LANES):
            cidx = perm_v[pl.ds(j * LANES, LANES)]

            def rbody(r, idx):
                val = plsc.load_gather(in_v, [idx])
                out_v[pl.ds(r * DIM + j * LANES, LANES)] = val
                return idx + DIM

            lax.fori_loop(0, CHUNK, rbody, cidx)
        pltpu.sync_copy(out_v, out_hbm.at[pl.ds(off, CHUNK * DIM)])
        return carry

    lax.fori_loop(0, ROWS_PER_W // CHUNK, chunk_body, 0)


@jax.jit
def _shuffle(x_flat, perm):
    mesh = plsc.VectorSubcoreMesh(core_axis_name="c", subcore_axis_name="s")
    f = pl.kernel(
        _shuffle_body,
        out_type=jax.ShapeDtypeStruct((ROWS * DIM,), jnp.float32),
        mesh=mesh,
        scratch_types=[
            pltpu.VMEM((DIM,), jnp.int32),
            pltpu.VMEM((CHUNK * DIM,), jnp.float32),
            pltpu.VMEM((CHUNK * DIM,), jnp.float32),
        ],
        compiler_params=pltpu.CompilerParams(needs_layout_passes=False),
    )
    return f(x_flat, perm)


def kernel(x, permutation):
    x_flat = x.reshape(ROWS * DIM)
    perm = permutation.astype(jnp.int32)
    out = _shuffle(x_flat, perm)
    return out.reshape(BATCH, SEQ, DIM)


# speedup vs baseline: 1.3413x; 1.3413x over previous
import jax
import jax.numpy as jnp
from jax import lax
from jax.experimental import pallas as pl
from jax.experimental.pallas import tpu as pltpu
from jax.experimental.pallas import tpu_sc as plsc

BATCH, SEQ, DIM = 4, 4096, 2048
ROWS = BATCH * SEQ
NC, NS = 2, 16
NW = NC * NS
ROWS_PER_W = ROWS // NW
CHUNK = 8
NCHUNK = ROWS_PER_W // CHUNK
LANES = 16
CELEMS = CHUNK * DIM


def _permute_chunk(in_v, out_v, perm_v):
    def jbody(j, carry):
        cidx = perm_v[pl.ds(j * LANES, LANES)]
        jo = j * LANES
        for r in range(CHUNK):
            val = plsc.load_gather(in_v, [cidx + (r * DIM)])
            out_v[pl.ds(jo + r * DIM, LANES)] = val
        return carry

    lax.fori_loop(0, DIM // LANES, jbody, 0)


def _shuffle_body(x_hbm, perm_hbm, out_hbm, perm_v, in_a, in_b, out_a, out_b,
                  si_a, si_b, so_a, so_b):
    wid = lax.axis_index("s") * NC + lax.axis_index("c")
    base = wid * (ROWS_PER_W * DIM)
    pltpu.sync_copy(perm_hbm, perm_v)

    def in_dma(c, buf, sem):
        return pltpu.make_async_copy(
            x_hbm.at[pl.ds(base + c * CELEMS, CELEMS)], buf, sem)

    def out_dma(c, buf, sem):
        return pltpu.make_async_copy(
            buf, out_hbm.at[pl.ds(base + c * CELEMS, CELEMS)], sem)

    in_dma(0, in_a, si_a).start()
    in_dma(1, in_b, si_b).start()

    def pair_body(cc, carry):
        c0 = 2 * cc
        in_dma(c0, in_a, si_a).wait()
        _permute_chunk(in_a, out_a, perm_v)

        @pl.when(cc < (NCHUNK // 2 - 1))
        def _():
            in_dma(c0 + 2, in_a, si_a).start()

        @pl.when(cc > 0)
        def _():
            out_dma(c0 - 2, out_a, so_a).wait()

        out_dma(c0, out_a, so_a).start()

        in_dma(c0 + 1, in_b, si_b).wait()
        _permute_chunk(in_b, out_b, perm_v)

        @pl.when(cc < (NCHUNK // 2 - 1))
        def _():
            in_dma(c0 + 3, in_b, si_b).start()

        @pl.when(cc > 0)
        def _():
            out_dma(c0 - 1, out_b, so_b).wait()

        out_dma(c0 + 1, out_b, so_b).start()
        return carry

    lax.fori_loop(0, NCHUNK // 2, pair_body, 0)
    out_dma(NCHUNK - 2, out_a, so_a).wait()
    out_dma(NCHUNK - 1, out_b, so_b).wait()


@jax.jit
def _shuffle(x_flat, perm):
    mesh = plsc.VectorSubcoreMesh(core_axis_name="c", subcore_axis_name="s")
    f = pl.kernel(
        _shuffle_body,
        out_type=jax.ShapeDtypeStruct((ROWS * DIM,), jnp.float32),
        mesh=mesh,
        scratch_types=[
            pltpu.VMEM((DIM,), jnp.int32),
            pltpu.VMEM((CELEMS,), jnp.float32),
            pltpu.VMEM((CELEMS,), jnp.float32),
            pltpu.VMEM((CELEMS,), jnp.float32),
            pltpu.VMEM((CELEMS,), jnp.float32),
            pltpu.SemaphoreType.DMA,
            pltpu.SemaphoreType.DMA,
            pltpu.SemaphoreType.DMA,
            pltpu.SemaphoreType.DMA,
        ],
        compiler_params=pltpu.CompilerParams(needs_layout_passes=False),
    )
    return f(x_flat, perm)


def kernel(x, permutation):
    x_flat = x.reshape(ROWS * DIM)
    perm = permutation.astype(jnp.int32)
    out = _shuffle(x_flat, perm)
    return out.reshape(BATCH, SEQ, DIM)


# speedup vs baseline: 2.2197x; 1.6549x over previous
import jax
import jax.numpy as jnp
from jax import lax
from jax.experimental import pallas as pl
from jax.experimental.pallas import tpu as pltpu
from jax.experimental.pallas import tpu_sc as plsc

BATCH, SEQ, DIM = 4, 4096, 2048
ROWS = BATCH * SEQ
NC, NS = 2, 16
NW = NC * NS
ROWS_PER_W = ROWS // NW
CHUNK = 8
NCHUNK = ROWS_PER_W // CHUNK
LANES = 16
CELEMS = CHUNK * DIM


def _permute_chunk(in_v, out_v, perm_v):
    @plsc.parallel_loop(0, DIM // LANES, unroll=2)
    def _(j):
        cidx = perm_v[pl.ds(j * LANES, LANES)]
        jo = j * LANES
        for r in range(CHUNK):
            val = plsc.load_gather(in_v, [cidx + (r * DIM)])
            out_v[pl.ds(jo + r * DIM, LANES)] = val


def _shuffle_body(x_hbm, perm_hbm, out_hbm, perm_v, in_a, in_b, out_a, out_b,
                  si_a, si_b, so_a, so_b):
    wid = lax.axis_index("s") * NC + lax.axis_index("c")
    base = wid * (ROWS_PER_W * DIM)
    pltpu.sync_copy(perm_hbm, perm_v)

    def in_dma(c, buf, sem):
        return pltpu.make_async_copy(
            x_hbm.at[pl.ds(base + c * CELEMS, CELEMS)], buf, sem)

    def out_dma(c, buf, sem):
        return pltpu.make_async_copy(
            buf, out_hbm.at[pl.ds(base + c * CELEMS, CELEMS)], sem)

    in_dma(0, in_a, si_a).start()
    in_dma(1, in_b, si_b).start()

    def pair_body(cc, carry):
        c0 = 2 * cc
        in_dma(c0, in_a, si_a).wait()
        _permute_chunk(in_a, out_a, perm_v)

        @pl.when(cc < (NCHUNK // 2 - 1))
        def _():
            in_dma(c0 + 2, in_a, si_a).start()

        @pl.when(cc > 0)
        def _():
            out_dma(c0 - 2, out_a, so_a).wait()

        out_dma(c0, out_a, so_a).start()

        in_dma(c0 + 1, in_b, si_b).wait()
        _permute_chunk(in_b, out_b, perm_v)

        @pl.when(cc < (NCHUNK // 2 - 1))
        def _():
            in_dma(c0 + 3, in_b, si_b).start()

        @pl.when(cc > 0)
        def _():
            out_dma(c0 - 1, out_b, so_b).wait()

        out_dma(c0 + 1, out_b, so_b).start()
        return carry

    lax.fori_loop(0, NCHUNK // 2, pair_body, 0)
    out_dma(NCHUNK - 2, out_a, so_a).wait()
    out_dma(NCHUNK - 1, out_b, so_b).wait()


@jax.jit
def _shuffle(x_flat, perm):
    mesh = plsc.VectorSubcoreMesh(core_axis_name="c", subcore_axis_name="s")
    f = pl.kernel(
        _shuffle_body,
        out_type=jax.ShapeDtypeStruct((ROWS * DIM,), jnp.float32),
        mesh=mesh,
        scratch_types=[
            pltpu.VMEM((DIM,), jnp.int32),
            pltpu.VMEM((CELEMS,), jnp.float32),
            pltpu.VMEM((CELEMS,), jnp.float32),
            pltpu.VMEM((CELEMS,), jnp.float32),
            pltpu.VMEM((CELEMS,), jnp.float32),
            pltpu.SemaphoreType.DMA,
            pltpu.SemaphoreType.DMA,
            pltpu.SemaphoreType.DMA,
            pltpu.SemaphoreType.DMA,
        ],
        compiler_params=pltpu.CompilerParams(needs_layout_passes=False),
    )
    return f(x_flat, perm)


def kernel(x, permutation):
    x_flat = x.reshape(ROWS * DIM)
    perm = permutation.astype(jnp.int32)
    out = _shuffle(x_flat, perm)
    return out.reshape(BATCH, SEQ, DIM)


# speedup vs baseline: 6.5185x; 2.9366x over previous
import jax
import jax.numpy as jnp
from jax import lax
from jax.experimental import pallas as pl
from jax.experimental.pallas import tpu as pltpu
from jax.experimental.pallas import tpu_sc as plsc

BATCH, SEQ, DIM = 4, 4096, 2048
ROWS = BATCH * SEQ
NC, NS = 2, 16
NW = NC * NS
ROWS_PER_W = ROWS // NW
CHUNK = 8
NCHUNK = ROWS_PER_W // CHUNK
LANES = 16


def _permute_chunk(in_v, out_v, perm_v):
    rvecs = [jnp.full((LANES,), r, jnp.int32) for r in range(CHUNK)]

    @plsc.parallel_loop(0, DIM // LANES, unroll=2)
    def _(j):
        cidx = perm_v[pl.ds(j * LANES, LANES)]
        jo = j * LANES
        for r in range(CHUNK):
            val = plsc.load_gather(in_v, [rvecs[r], cidx])
            out_v[r, pl.ds(jo, LANES)] = val


def _shuffle_body(x_hbm, perm_hbm, out_hbm, perm_v, in_a, in_b, out_a, out_b,
                  si_a, si_b, so_a, so_b):
    wid = lax.axis_index("s") * NC + lax.axis_index("c")
    base = wid * ROWS_PER_W
    pltpu.sync_copy(perm_hbm, perm_v)

    def in_dma(c, buf, sem):
        return pltpu.make_async_copy(
            x_hbm.at[pl.ds(base + c * CHUNK, CHUNK)], buf, sem)

    def out_dma(c, buf, sem):
        return pltpu.make_async_copy(
            buf, out_hbm.at[pl.ds(base + c * CHUNK, CHUNK)], sem)

    in_dma(0, in_a, si_a).start()
    in_dma(1, in_b, si_b).start()

    def pair_body(cc, carry):
        c0 = 2 * cc
        in_dma(c0, in_a, si_a).wait()
        _permute_chunk(in_a, out_a, perm_v)

        @pl.when(cc < (NCHUNK // 2 - 1))
        def _():
            in_dma(c0 + 2, in_a, si_a).start()

        @pl.when(cc > 0)
        def _():
            out_dma(c0 - 2, out_a, so_a).wait()

        out_dma(c0, out_a, so_a).start()

        in_dma(c0 + 1, in_b, si_b).wait()
        _permute_chunk(in_b, out_b, perm_v)

        @pl.when(cc < (NCHUNK // 2 - 1))
        def _():
            in_dma(c0 + 3, in_b, si_b).start()

        @pl.when(cc > 0)
        def _():
            out_dma(c0 - 1, out_b, so_b).wait()

        out_dma(c0 + 1, out_b, so_b).start()
        return carry

    lax.fori_loop(0, NCHUNK // 2, pair_body, 0)
    out_dma(NCHUNK - 2, out_a, so_a).wait()
    out_dma(NCHUNK - 1, out_b, so_b).wait()


@jax.jit
def _shuffle(x2, perm):
    mesh = plsc.VectorSubcoreMesh(core_axis_name="c", subcore_axis_name="s")
    f = pl.kernel(
        _shuffle_body,
        out_type=jax.ShapeDtypeStruct((ROWS, DIM), jnp.float32),
        mesh=mesh,
        scratch_types=[
            pltpu.VMEM((DIM,), jnp.int32),
            pltpu.VMEM((CHUNK, DIM), jnp.float32),
            pltpu.VMEM((CHUNK, DIM), jnp.float32),
            pltpu.VMEM((CHUNK, DIM), jnp.float32),
            pltpu.VMEM((CHUNK, DIM), jnp.float32),
            pltpu.SemaphoreType.DMA,
            pltpu.SemaphoreType.DMA,
            pltpu.SemaphoreType.DMA,
            pltpu.SemaphoreType.DMA,
        ],
        compiler_params=pltpu.CompilerParams(needs_layout_passes=False),
    )
    return f(x2, perm)


def kernel(x, permutation):
    x2 = x.reshape(ROWS, DIM)
    perm = permutation.astype(jnp.int32)
    out = _shuffle(x2, perm)
    return out.reshape(BATCH, SEQ, DIM)


# speedup vs baseline: 6.7692x; 1.0385x over previous
import jax
import jax.numpy as jnp
from jax import lax
from jax.experimental import pallas as pl
from jax.experimental.pallas import tpu as pltpu
from jax.experimental.pallas import tpu_sc as plsc

BATCH, SEQ, DIM = 4, 4096, 2048
ROWS = BATCH * SEQ
NC, NS = 2, 16
NW = NC * NS
ROWS_PER_W = ROWS // NW
CHUNK = 4
NCHUNK = ROWS_PER_W // CHUNK
LANES = 16


def _permute_chunk(in_v, out_v, perm_v):
    rvecs = [jnp.full((LANES,), r, jnp.int32) for r in range(CHUNK)]

    @plsc.parallel_loop(0, DIM // LANES, unroll=2)
    def _(j):
        cidx = perm_v[pl.ds(j * LANES, LANES)]
        jo = j * LANES
        for r in range(CHUNK):
            val = plsc.load_gather(in_v, [rvecs[r], cidx])
            out_v[r, pl.ds(jo, LANES)] = val


NBUF = 4


def _shuffle_body(x_hbm, perm_hbm, out_hbm, perm_v,
                  in_0, in_1, in_2, in_3, out_0, out_1, out_2, out_3,
                  si_0, si_1, si_2, si_3, so_0, so_1, so_2, so_3):
    in_bufs = (in_0, in_1, in_2, in_3)
    out_bufs = (out_0, out_1, out_2, out_3)
    si = (si_0, si_1, si_2, si_3)
    so = (so_0, so_1, so_2, so_3)

    wid = lax.axis_index("s") * NC + lax.axis_index("c")
    base = wid * ROWS_PER_W
    pltpu.sync_copy(perm_hbm, perm_v)

    def in_dma(c, k):
        return pltpu.make_async_copy(
            x_hbm.at[pl.ds(base + c * CHUNK, CHUNK)], in_bufs[k], si[k])

    def out_dma(c, k):
        return pltpu.make_async_copy(
            out_bufs[k], out_hbm.at[pl.ds(base + c * CHUNK, CHUNK)], so[k])

    for k in range(NBUF):
        in_dma(k, k).start()

    def ring_body(cc, carry):
        c0 = NBUF * cc
        for k in range(NBUF):
            c = c0 + k
            in_dma(c, k).wait()

            @pl.when(cc > 0)
            def _():
                out_dma(c - NBUF, k).wait()

            _permute_chunk(in_bufs[k], out_bufs[k], perm_v)

            @pl.when(cc < (NCHUNK // NBUF - 1))
            def _():
                in_dma(c + NBUF, k).start()

            out_dma(c, k).start()
        return carry

    lax.fori_loop(0, NCHUNK // NBUF, ring_body, 0)
    for k in range(NBUF):
        out_dma(NCHUNK - NBUF + k, k).wait()


@jax.jit
def _shuffle(x2, perm):
    mesh = plsc.VectorSubcoreMesh(core_axis_name="c", subcore_axis_name="s")
    f = pl.kernel(
        _shuffle_body,
        out_type=jax.ShapeDtypeStruct((ROWS, DIM), jnp.float32),
        mesh=mesh,
        scratch_types=(
            [pltpu.VMEM((DIM,), jnp.int32)]
            + [pltpu.VMEM((CHUNK, DIM), jnp.float32)] * (2 * NBUF)
            + [pltpu.SemaphoreType.DMA] * (2 * NBUF)
        ),
        compiler_params=pltpu.CompilerParams(needs_layout_passes=False),
    )
    return f(x2, perm)


def kernel(x, permutation):
    x2 = x.reshape(ROWS, DIM)
    perm = permutation.astype(jnp.int32)
    out = _shuffle(x2, perm)
    return out.reshape(BATCH, SEQ, DIM)


# speedup vs baseline: 6.7945x; 1.0037x over previous
import jax
import jax.numpy as jnp
from jax import lax
from jax.experimental import pallas as pl
from jax.experimental.pallas import tpu as pltpu
from jax.experimental.pallas import tpu_sc as plsc

BATCH, SEQ, DIM = 4, 4096, 2048
ROWS = BATCH * SEQ
NC, NS = 2, 16
NW = NC * NS
ROWS_PER_W = ROWS // NW
CHUNK = 4
NCHUNK = ROWS_PER_W // CHUNK
LANES = 16


def _permute_chunk(in_v, out_v, perm_v):
    rvecs = [jnp.full((LANES,), r, jnp.int32) for r in range(CHUNK)]

    @plsc.parallel_loop(0, DIM // LANES, unroll=4)
    def _(j):
        cidx = perm_v[pl.ds(j * LANES, LANES)]
        jo = j * LANES
        for r in range(CHUNK):
            val = plsc.load_gather(in_v, [rvecs[r], cidx])
            out_v[r, pl.ds(jo, LANES)] = val


NBUF = 4


def _shuffle_body(x_hbm, perm_hbm, out_hbm, perm_v,
                  in_0, in_1, in_2, in_3, out_0, out_1, out_2, out_3,
                  si_0, si_1, si_2, si_3, so_0, so_1, so_2, so_3):
    in_bufs = (in_0, in_1, in_2, in_3)
    out_bufs = (out_0, out_1, out_2, out_3)
    si = (si_0, si_1, si_2, si_3)
    so = (so_0, so_1, so_2, so_3)

    wid = lax.axis_index("s") * NC + lax.axis_index("c")
    base = wid * ROWS_PER_W
    pltpu.sync_copy(perm_hbm, perm_v)

    def in_dma(c, k):
        return pltpu.make_async_copy(
            x_hbm.at[pl.ds(base + c * CHUNK, CHUNK)], in_bufs[k], si[k])

    def out_dma(c, k):
        return pltpu.make_async_copy(
            out_bufs[k], out_hbm.at[pl.ds(base + c * CHUNK, CHUNK)], so[k])

    for k in range(NBUF):
        in_dma(k, k).start()

    def ring_body(cc, carry):
        c0 = NBUF * cc
        for k in range(NBUF):
            c = c0 + k
            in_dma(c, k).wait()

            @pl.when(cc > 0)
            def _():
                out_dma(c - NBUF, k).wait()

            _permute_chunk(in_bufs[k], out_bufs[k], perm_v)

            @pl.when(cc < (NCHUNK // NBUF - 1))
            def _():
                in_dma(c + NBUF, k).start()

            out_dma(c, k).start()
        return carry

    lax.fori_loop(0, NCHUNK // NBUF, ring_body, 0)
    for k in range(NBUF):
        out_dma(NCHUNK - NBUF + k, k).wait()


@jax.jit
def _shuffle(x2, perm):
    mesh = plsc.VectorSubcoreMesh(core_axis_name="c", subcore_axis_name="s")
    f = pl.kernel(
        _shuffle_body,
        out_type=jax.ShapeDtypeStruct((ROWS, DIM), jnp.float32),
        mesh=mesh,
        scratch_types=(
            [pltpu.VMEM((DIM,), jnp.int32)]
            + [pltpu.VMEM((CHUNK, DIM), jnp.float32)] * (2 * NBUF)
            + [pltpu.SemaphoreType.DMA] * (2 * NBUF)
        ),
        compiler_params=pltpu.CompilerParams(needs_layout_passes=False),
    )
    return f(x2, perm)


def kernel(x, permutation):
    x2 = x.reshape(ROWS, DIM)
    perm = permutation.astype(jnp.int32)
    out = _shuffle(x2, perm)
    return out.reshape(BATCH, SEQ, DIM)
